# Initial kernel scaffold; baseline (speedup 1.0000x reference)
import functools

import jax
import jax.numpy as jnp
from jax import lax
from jax.experimental import pallas as pl
from jax.experimental.pallas import tpu as pltpu
from jax.experimental.pallas import tpu_sc as plsc

BATCH = 16384
FIELDS = 26
EMBED_DIM = 32

_B = BATCH * FIELDS
_NC = 2
_NS = 16
_NW = _NC * _NS
_BPW = _B // _NW
_CHUNK = 1024
_NCHUNK = _BPW // _CHUNK

_mesh = plsc.VectorSubcoreMesh(core_axis_name="c", subcore_axis_name="s")


@functools.partial(
    pl.kernel,
    mesh=_mesh,
    out_type=jax.ShapeDtypeStruct((_B, EMBED_DIM), jnp.float32),
    scratch_types=[
        pltpu.VMEM((_BPW,), jnp.int32),
        pltpu.VMEM((_CHUNK, EMBED_DIM), jnp.float32),
        pltpu.VMEM((_CHUNK, EMBED_DIM), jnp.float32),
        pltpu.SemaphoreType.DMA,
        pltpu.SemaphoreType.DMA,
    ],
)
def _gather_all(idx_hbm, table_hbm, out_hbm, idx_v, rows_a, rows_b, sem_a, sem_b):
    wid = lax.axis_index("s") * _NC + lax.axis_index("c")
    base = wid * _BPW
    pltpu.sync_copy(idx_hbm.at[pl.ds(base, _BPW)], idx_v)

    pltpu.async_copy(table_hbm.at[idx_v.at[pl.ds(0, _CHUNK)]], rows_a, sem_a)

    def body(c, carry):
        even = lax.rem(c, 2) == 0
        @pl.when(c + 1 < _NCHUNK)
        def _():
            nxt = (c + 1) * _CHUNK

            @pl.when(even)
            def _():
                pltpu.async_copy(
                    table_hbm.at[idx_v.at[pl.ds(nxt, _CHUNK)]], rows_b, sem_b)

            @pl.when(jnp.logical_not(even))
            def _():
                pltpu.async_copy(
                    table_hbm.at[idx_v.at[pl.ds(nxt, _CHUNK)]], rows_a, sem_a)

        off = base + c * _CHUNK

        @pl.when(even)
        def _():
            pltpu.make_async_copy(
                table_hbm.at[idx_v.at[pl.ds(0, _CHUNK)]], rows_a, sem_a).wait()
            pltpu.sync_copy(rows_a, out_hbm.at[pl.ds(off, _CHUNK)])

        @pl.when(jnp.logical_not(even))
        def _():
            pltpu.make_async_copy(
                table_hbm.at[idx_v.at[pl.ds(0, _CHUNK)]], rows_b, sem_b).wait()
            pltpu.sync_copy(rows_b, out_hbm.at[pl.ds(off, _CHUNK)])

        return carry

    lax.fori_loop(0, _NCHUNK, body, 0)


def kernel(cat_idx, table):
    flat_idx = cat_idx.reshape(_B).astype(jnp.int32)
    out = _gather_all(flat_idx, table)
    return out.reshape(BATCH, FIELDS, EMBED_DIM)


# speedup vs baseline: 1.5772x; 1.5772x over previous
import functools

import jax
import jax.numpy as jnp
from jax import lax
from jax.experimental import pallas as pl
from jax.experimental.pallas import tpu as pltpu
from jax.experimental.pallas import tpu_sc as plsc

BATCH = 16384
FIELDS = 26
EMBED_DIM = 32

_B = BATCH * FIELDS
_NC = 2
_NS = 16
_NW = _NC * _NS
_BPW = _B // _NW
_CHUNK = 1024
_NCHUNK = _BPW // _CHUNK

_mesh = plsc.VectorSubcoreMesh(core_axis_name="c", subcore_axis_name="s")


@functools.partial(
    pl.kernel,
    mesh=_mesh,
    compiler_params=pltpu.CompilerParams(use_tc_tiling_on_sc=False),
    out_type=jax.ShapeDtypeStruct((_B, EMBED_DIM), jnp.float32),
    scratch_types=[
        pltpu.VMEM((_BPW,), jnp.int32),
        pltpu.VMEM((_CHUNK, EMBED_DIM), jnp.float32),
        pltpu.VMEM((_CHUNK, EMBED_DIM), jnp.float32),
        pltpu.SemaphoreType.DMA,
        pltpu.SemaphoreType.DMA,
    ],
)
def _gather_all(idx_hbm, table_hbm, out_hbm, idx_v, rows_a, rows_b, sem_a, sem_b):
    wid = lax.axis_index("s") * _NC + lax.axis_index("c")
    base = wid * _BPW
    pltpu.sync_copy(idx_hbm.at[pl.ds(base, _BPW)], idx_v)

    pltpu.async_copy(table_hbm.at[idx_v.at[pl.ds(0, _CHUNK)]], rows_a, sem_a)

    def body(c, carry):
        even = lax.rem(c, 2) == 0
        @pl.when(c + 1 < _NCHUNK)
        def _():
            nxt = (c + 1) * _CHUNK

            @pl.when(even)
            def _():
                pltpu.async_copy(
                    table_hbm.at[idx_v.at[pl.ds(nxt, _CHUNK)]], rows_b, sem_b)

            @pl.when(jnp.logical_not(even))
            def _():
                pltpu.async_copy(
                    table_hbm.at[idx_v.at[pl.ds(nxt, _CHUNK)]], rows_a, sem_a)

        off = base + c * _CHUNK

        @pl.when(even)
        def _():
            pltpu.make_async_copy(
                table_hbm.at[idx_v.at[pl.ds(0, _CHUNK)]], rows_a, sem_a).wait()
            pltpu.sync_copy(rows_a, out_hbm.at[pl.ds(off, _CHUNK)])

        @pl.when(jnp.logical_not(even))
        def _():
            pltpu.make_async_copy(
                table_hbm.at[idx_v.at[pl.ds(0, _CHUNK)]], rows_b, sem_b).wait()
            pltpu.sync_copy(rows_b, out_hbm.at[pl.ds(off, _CHUNK)])

        return carry

    lax.fori_loop(0, _NCHUNK, body, 0)


def kernel(cat_idx, table):
    flat_idx = cat_idx.reshape(_B).astype(jnp.int32)
    out = _gather_all(flat_idx, table)
    return out.reshape(BATCH, FIELDS, EMBED_DIM)


# speedup vs baseline: 1.5775x; 1.0002x over previous
import functools

import jax
import jax.numpy as jnp
from jax import lax
from jax.experimental import pallas as pl
from jax.experimental.pallas import tpu as pltpu
from jax.experimental.pallas import tpu_sc as plsc

BATCH = 16384
FIELDS = 26
EMBED_DIM = 32

_B = BATCH * FIELDS
_NC = 2
_NS = 16
_NW = _NC * _NS
_BPW = _B // _NW
_CHUNK = 416
_NCHUNK = _BPW // _CHUNK
_NBUF = 8

_mesh = plsc.VectorSubcoreMesh(core_axis_name="c", subcore_axis_name="s")


@functools.partial(
    pl.kernel,
    mesh=_mesh,
    compiler_params=pltpu.CompilerParams(use_tc_tiling_on_sc=False),
    out_type=jax.ShapeDtypeStruct((_B, EMBED_DIM), jnp.float32),
    scratch_types=[
        pltpu.VMEM((_BPW,), jnp.int32),
        tuple(pltpu.VMEM((_CHUNK, EMBED_DIM), jnp.float32) for _ in range(_NBUF)),
        tuple(pltpu.SemaphoreType.DMA for _ in range(_NBUF)),
        tuple(pltpu.SemaphoreType.DMA for _ in range(_NBUF)),
    ],
)
def _gather_all(idx_hbm, table_hbm, out_hbm, idx_v, rows, gsem, wsem):
    wid = lax.axis_index("s") * _NC + lax.axis_index("c")
    base = wid * _BPW
    pltpu.sync_copy(idx_hbm.at[pl.ds(base, _BPW)], idx_v)

    def start_gather(c, b):
        pltpu.async_copy(
            table_hbm.at[idx_v.at[pl.ds(c * _CHUNK, _CHUNK)]], rows[b], gsem[b])

    def wait_gather(b):
        pltpu.make_async_copy(
            table_hbm.at[idx_v.at[pl.ds(0, _CHUNK)]], rows[b], gsem[b]).wait()

    def start_write(c, b):
        pltpu.async_copy(rows[b], out_hbm.at[pl.ds(base + c * _CHUNK, _CHUNK)],
                         wsem[b])

    def wait_write(b):
        pltpu.make_async_copy(
            rows[b], out_hbm.at[pl.ds(0, _CHUNK)], wsem[b]).wait()

    for b in range(_NBUF - 1):
        start_gather(b, b)

    @pl.loop(0, _NCHUNK, step=_NBUF)
    def _ring(outer):
        for b in range(_NBUF):
            c = outer + b
            g = c + _NBUF - 1
            bg = (b + _NBUF - 1) % _NBUF

            @pl.when(g < _NCHUNK)
            def _():
                @pl.when(g >= _NBUF)
                def _():
                    wait_write(bg)
                start_gather(g, bg)

            wait_gather(b)
            start_write(c, b)

    for b in range(_NBUF):
        wait_write(b)


def kernel(cat_idx, table):
    flat_idx = cat_idx.reshape(_B).astype(jnp.int32)
    out = _gather_all(flat_idx, table)
    return out.reshape(BATCH, FIELDS, EMBED_DIM)
